# baseline (speedup 1.0000x reference)
import functools

import jax
import jax.numpy as jnp
from jax import lax
from jax.experimental import pallas as pl
from jax.experimental.pallas import tpu as pltpu
from jax.experimental.pallas import tpu_sc as plsc

_NC = 2
_NS = 16
_NW = _NC * _NS
_CHUNK = 16
_NBUF = 5


def _emb_body(bpw, nchunk,
              idx_hbm, table_hbm, out_hbm,
              idx_v, rows, gs, ss):
    wid = lax.axis_index("s") * _NC + lax.axis_index("c")
    base = wid * bpw

    pltpu.sync_copy(idx_hbm.at[wid], idx_v)
    pltpu.sync_copy(idx_hbm.at[wid], idx_v)

    def gather(chunk, b):
        src = table_hbm.at[idx_v.at[chunk]]
        return pltpu.make_async_copy(src, rows[b], gs[b])

    def store(chunk, b):
        dst = out_hbm.at[pl.ds(base + chunk * _CHUNK, _CHUNK)]
        return pltpu.make_async_copy(rows[b], dst, ss[b])

    def step(c, j):
        bp = (j + _NBUF - 1) % _NBUF
        bpp = (j + _NBUF - 2) % _NBUF
        bn = (j + 2) % _NBUF
        gather(c, j).wait()

        @pl.when(c >= 1)
        def _():
            store(c - 1, bp).start()

        @pl.when(c >= 2)
        def _():
            store(c - 2, bpp).wait()

        @pl.when(c + 2 < nchunk)
        def _():
            gather(c + 2, bn).start()

    gather(0, 0).start()
    gather(1, 1).start()

    ngroup = nchunk // _NBUF

    def group(g, _):
        for j in range(_NBUF):
            step(_NBUF * g + j, j)
        return None

    lax.fori_loop(0, ngroup, group, None, unroll=False)

    for c in range(_NBUF * ngroup, nchunk):
        step(c, c % _NBUF)
    pltpu.sync_copy(idx_hbm.at[wid], idx_v)
    store(nchunk - 1, (nchunk - 1) % _NBUF).start()
    store(nchunk - 2, (nchunk - 2) % _NBUF).wait()
    store(nchunk - 1, (nchunk - 1) % _NBUF).wait()


def kernel(pos_ids, table):
    batch, seq = pos_ids.shape
    dim = table.shape[1]
    total = batch * seq
    bpw = total // _NW
    nchunk = bpw // _CHUNK

    flat_ids = pos_ids.reshape(_NW, nchunk, _CHUNK).astype(jnp.int32)

    mesh = plsc.VectorSubcoreMesh(core_axis_name="c", subcore_axis_name="s")
    body = functools.partial(_emb_body, bpw, nchunk)
    out = pl.kernel(
        body,
        out_type=jax.ShapeDtypeStruct((total, dim), jnp.float32),
        mesh=mesh,
        scratch_types=[
            pltpu.VMEM((nchunk, _CHUNK), jnp.int32),
            [pltpu.VMEM((_CHUNK, dim), jnp.float32) for _ in range(_NBUF)],
            [pltpu.SemaphoreType.DMA for _ in range(_NBUF)],
            [pltpu.SemaphoreType.DMA for _ in range(_NBUF)],
        ],
    )(flat_ids, table)
    return out.reshape(batch, seq, dim)
